# baseline (speedup 1.0000x reference)
import jax
import jax.numpy as jnp
from jax.experimental import pallas as pl


_DN = (((1,), (1,)), ((), ()))


def _tile_kernel(s0_ref, e0_ref, c0_ref, s1_ref, e1_ref, c1_ref, x_ref,
                 l0_ref, l1_ref, p0_ref, p1_ref, pred_ref, ent_ref):
    bN = s0_ref.shape[0]
    C = x_ref.shape[1]
    cls = jax.lax.broadcasted_iota(jnp.int32, (bN, C), 1)
    m0 = ((cls >= s0_ref[...]) & (cls < e0_ref[...])).astype(jnp.bfloat16)
    m1 = ((cls >= s1_ref[...]) & (cls < e1_ref[...])).astype(jnp.bfloat16)
    x = x_ref[...]
    xhi = x.astype(jnp.bfloat16)
    xlo = (x - xhi.astype(jnp.float32)).astype(jnp.bfloat16)
    f32 = jnp.float32

    def mm(m, v):
        return jax.lax.dot_general(m, v, _DN, preferred_element_type=f32)

    l0 = (mm(m0, xhi) + mm(m0, xlo)) / c0_ref[...]
    l1 = (mm(m1, xhi) + mm(m1, xlo)) / c1_ref[...]
    d = l1 - l0
    p0 = jax.nn.sigmoid(-d)
    p1 = jax.nn.sigmoid(d)
    l0_ref[...] = l0
    l1_ref[...] = l1
    p0_ref[...] = p0
    p1_ref[...] = p1
    pred_ref[...] = (d > 0).astype(jnp.int32)
    ent_ref[...] = -(p0 * jnp.log(p0) + p1 * jnp.log(p1))


def kernel(outputs, gather_idx, segment_ids, counts):
    B, C = outputs.shape
    S = counts.shape[0]
    N = S // 2
    del segment_ids

    cnt_i = counts.astype(jnp.int32)
    offsets = jnp.concatenate(
        [jnp.zeros((1,), jnp.int32), jnp.cumsum(cnt_i)[:-1]])
    starts = gather_idx[offsets]
    ends = starts + cnt_i

    s0 = starts[0::2][:, None]
    e0 = ends[0::2][:, None]
    s1 = starts[1::2][:, None]
    e1 = ends[1::2][:, None]
    c0 = counts[0::2][:, None]
    c1 = counts[1::2][:, None]

    bN = 128
    bB = 512
    grid = (B // bB, pl.cdiv(N, bN))

    seg_spec = pl.BlockSpec((bN, 1), lambda j, i: (i, 0))
    out_spec = pl.BlockSpec((bN, bB), lambda j, i: (i, j))
    f32 = jnp.float32
    l0, l1, p0, p1, preds, ent = pl.pallas_call(
        _tile_kernel,
        grid=grid,
        in_specs=[seg_spec, seg_spec, seg_spec, seg_spec, seg_spec, seg_spec,
                  pl.BlockSpec((bB, C), lambda j, i: (j, 0))],
        out_specs=[out_spec] * 6,
        out_shape=[
            jax.ShapeDtypeStruct((N, B), f32),
            jax.ShapeDtypeStruct((N, B), f32),
            jax.ShapeDtypeStruct((N, B), f32),
            jax.ShapeDtypeStruct((N, B), f32),
            jax.ShapeDtypeStruct((N, B), jnp.int32),
            jax.ShapeDtypeStruct((N, B), f32),
        ],
    )(s0, e0, c0, s1, e1, c1, outputs)

    node_logits = jnp.stack([l0, l1], axis=-1)
    probs = jnp.stack([p0, p1], axis=-1)
    return node_logits, probs, preds, ent


# speedup vs baseline: 1.3278x; 1.3278x over previous
import jax
import jax.numpy as jnp
from jax.experimental import pallas as pl


def _tile_kernel(s0_ref, e0_ref, c0_ref, s1_ref, e1_ref, c1_ref, xT_ref,
                 l0_ref, l1_ref, p0_ref, p1_ref, pred_ref, ent_ref):
    bN = s0_ref.shape[0]
    C = xT_ref.shape[0]
    cls = jax.lax.broadcasted_iota(jnp.int32, (bN, C), 1)
    m0 = ((cls >= s0_ref[...]) & (cls < e0_ref[...])).astype(jnp.bfloat16)
    m1 = ((cls >= s1_ref[...]) & (cls < e1_ref[...])).astype(jnp.bfloat16)
    x = xT_ref[...]
    xhi = x.astype(jnp.bfloat16)
    xlo = (x - xhi.astype(jnp.float32)).astype(jnp.bfloat16)
    f32 = jnp.float32
    l0 = (jnp.dot(m0, xhi, preferred_element_type=f32)
          + jnp.dot(m0, xlo, preferred_element_type=f32)) / c0_ref[...]
    l1 = (jnp.dot(m1, xhi, preferred_element_type=f32)
          + jnp.dot(m1, xlo, preferred_element_type=f32)) / c1_ref[...]
    d = l1 - l0
    p0 = jax.nn.sigmoid(-d)
    p1 = jax.nn.sigmoid(d)
    l0_ref[...] = l0
    l1_ref[...] = l1
    p0_ref[...] = p0
    p1_ref[...] = p1
    pred_ref[...] = (d > 0).astype(jnp.int32)
    ent_ref[...] = -(p0 * jnp.log(p0) + p1 * jnp.log(p1))


def kernel(outputs, gather_idx, segment_ids, counts):
    B, C = outputs.shape
    S = counts.shape[0]
    N = S // 2
    del segment_ids

    cnt_i = counts.astype(jnp.int32)
    offsets = jnp.concatenate(
        [jnp.zeros((1,), jnp.int32), jnp.cumsum(cnt_i)[:-1]])
    starts = gather_idx[offsets]
    ends = starts + cnt_i

    s0 = starts[0::2][:, None]
    e0 = ends[0::2][:, None]
    s1 = starts[1::2][:, None]
    e1 = ends[1::2][:, None]
    c0 = counts[0::2][:, None]
    c1 = counts[1::2][:, None]

    xT = outputs.T

    bN = 128
    bB = 1024
    grid = (B // bB, pl.cdiv(N, bN))

    seg_spec = pl.BlockSpec((bN, 1), lambda j, i: (i, 0))
    out_spec = pl.BlockSpec((bN, bB), lambda j, i: (i, j))
    f32 = jnp.float32
    l0, l1, p0, p1, preds, ent = pl.pallas_call(
        _tile_kernel,
        grid=grid,
        in_specs=[seg_spec, seg_spec, seg_spec, seg_spec, seg_spec, seg_spec,
                  pl.BlockSpec((C, bB), lambda j, i: (0, j))],
        out_specs=[out_spec] * 6,
        out_shape=[
            jax.ShapeDtypeStruct((N, B), f32),
            jax.ShapeDtypeStruct((N, B), f32),
            jax.ShapeDtypeStruct((N, B), f32),
            jax.ShapeDtypeStruct((N, B), f32),
            jax.ShapeDtypeStruct((N, B), jnp.int32),
            jax.ShapeDtypeStruct((N, B), f32),
        ],
    )(s0, e0, c0, s1, e1, c1, xT)

    node_logits = jnp.stack([l0, l1], axis=-1)
    probs = jnp.stack([p0, p1], axis=-1)
    return node_logits, probs, preds, ent


# speedup vs baseline: 1.4199x; 1.0693x over previous
import jax
import jax.numpy as jnp
from jax.experimental import pallas as pl


def _tile_kernel(s0_ref, e0_ref, c0_ref, s1_ref, e1_ref, c1_ref, xT_ref,
                 l0_ref, l1_ref, p0_ref, p1_ref, pred_ref, ent_ref):
    bN = s0_ref.shape[0]
    C = xT_ref.shape[0]
    cls = jax.lax.broadcasted_iota(jnp.int32, (bN, C), 1)
    m0 = ((cls >= s0_ref[...]) & (cls < e0_ref[...])).astype(jnp.bfloat16)
    m1 = ((cls >= s1_ref[...]) & (cls < e1_ref[...])).astype(jnp.bfloat16)
    x = xT_ref[...]
    xhi = x.astype(jnp.bfloat16)
    xlo = (x - xhi.astype(jnp.float32)).astype(jnp.bfloat16)
    f32 = jnp.float32
    l0 = (jnp.dot(m0, xhi, preferred_element_type=f32)
          + jnp.dot(m0, xlo, preferred_element_type=f32)) / c0_ref[...]
    l1 = (jnp.dot(m1, xhi, preferred_element_type=f32)
          + jnp.dot(m1, xlo, preferred_element_type=f32)) / c1_ref[...]
    d = l1 - l0
    p0 = jax.nn.sigmoid(-d)
    p1 = jax.nn.sigmoid(d)
    l0_ref[...] = l0
    l1_ref[...] = l1
    p0_ref[...] = p0
    p1_ref[...] = p1
    pred_ref[...] = (d > 0).astype(jnp.int32)
    ent_ref[...] = -(p0 * jnp.log(p0) + p1 * jnp.log(p1))


def kernel(outputs, gather_idx, segment_ids, counts):
    B, C = outputs.shape
    S = counts.shape[0]
    N = S // 2
    del segment_ids

    cnt_i = counts.astype(jnp.int32)
    offsets = jnp.concatenate(
        [jnp.zeros((1,), jnp.int32), jnp.cumsum(cnt_i)[:-1]])
    starts = gather_idx[offsets]
    ends = starts + cnt_i

    s0 = starts[0::2][:, None]
    e0 = ends[0::2][:, None]
    s1 = starts[1::2][:, None]
    e1 = ends[1::2][:, None]
    c0 = counts[0::2][:, None]
    c1 = counts[1::2][:, None]

    xT = outputs.T

    bN = 256
    bB = 1024
    grid = (B // bB, pl.cdiv(N, bN))

    seg_spec = pl.BlockSpec((bN, 1), lambda j, i: (i, 0))
    out_spec = pl.BlockSpec((bN, bB), lambda j, i: (i, j))
    f32 = jnp.float32
    l0, l1, p0, p1, preds, ent = pl.pallas_call(
        _tile_kernel,
        grid=grid,
        in_specs=[seg_spec, seg_spec, seg_spec, seg_spec, seg_spec, seg_spec,
                  pl.BlockSpec((C, bB), lambda j, i: (0, j))],
        out_specs=[out_spec] * 6,
        out_shape=[
            jax.ShapeDtypeStruct((N, B), f32),
            jax.ShapeDtypeStruct((N, B), f32),
            jax.ShapeDtypeStruct((N, B), f32),
            jax.ShapeDtypeStruct((N, B), f32),
            jax.ShapeDtypeStruct((N, B), jnp.int32),
            jax.ShapeDtypeStruct((N, B), f32),
        ],
    )(s0, e0, c0, s1, e1, c1, xT)

    node_logits = jnp.stack([l0, l1], axis=-1)
    probs = jnp.stack([p0, p1], axis=-1)
    return node_logits, probs, preds, ent


# speedup vs baseline: 1.4508x; 1.0218x over previous
import jax
import jax.numpy as jnp
from jax.experimental import pallas as pl


def _tile_kernel(s0_ref, e0_ref, c0_ref, s1_ref, e1_ref, c1_ref, xT_ref,
                 l0_ref, l1_ref, p0_ref, p1_ref, pred_ref, ent_ref):
    bN = s0_ref.shape[0]
    C = xT_ref.shape[0]
    cls = jax.lax.broadcasted_iota(jnp.int32, (bN, C), 1)
    m0 = ((cls >= s0_ref[...]) & (cls < e0_ref[...])).astype(jnp.bfloat16)
    m1 = ((cls >= s1_ref[...]) & (cls < e1_ref[...])).astype(jnp.bfloat16)
    x = xT_ref[...]
    xhi = x.astype(jnp.bfloat16)
    xlo = (x - xhi.astype(jnp.float32)).astype(jnp.bfloat16)
    f32 = jnp.float32
    l0 = (jnp.dot(m0, xhi, preferred_element_type=f32)
          + jnp.dot(m0, xlo, preferred_element_type=f32)) / c0_ref[...]
    l1 = (jnp.dot(m1, xhi, preferred_element_type=f32)
          + jnp.dot(m1, xlo, preferred_element_type=f32)) / c1_ref[...]
    d = l1 - l0
    p0 = jax.nn.sigmoid(-d)
    p1 = jax.nn.sigmoid(d)
    l0_ref[...] = l0
    l1_ref[...] = l1
    p0_ref[...] = p0
    p1_ref[...] = p1
    pred_ref[...] = (d > 0).astype(jnp.int32)
    ent_ref[...] = -(p0 * jnp.log(p0) + p1 * jnp.log(p1))


def kernel(outputs, gather_idx, segment_ids, counts):
    B, C = outputs.shape
    S = counts.shape[0]
    N = S // 2
    del segment_ids

    cnt_i = counts.astype(jnp.int32)
    offsets = jnp.concatenate(
        [jnp.zeros((1,), jnp.int32), jnp.cumsum(cnt_i)[:-1]])
    starts = gather_idx[offsets]
    ends = starts + cnt_i

    s0 = starts[0::2][:, None]
    e0 = ends[0::2][:, None]
    s1 = starts[1::2][:, None]
    e1 = ends[1::2][:, None]
    c0 = counts[0::2][:, None]
    c1 = counts[1::2][:, None]

    xT = outputs.T

    bN = 256
    bB = 2048
    grid = (B // bB, pl.cdiv(N, bN))

    seg_spec = pl.BlockSpec((bN, 1), lambda j, i: (i, 0))
    out_spec = pl.BlockSpec((bN, bB), lambda j, i: (i, j))
    f32 = jnp.float32
    l0, l1, p0, p1, preds, ent = pl.pallas_call(
        _tile_kernel,
        grid=grid,
        in_specs=[seg_spec, seg_spec, seg_spec, seg_spec, seg_spec, seg_spec,
                  pl.BlockSpec((C, bB), lambda j, i: (0, j))],
        out_specs=[out_spec] * 6,
        out_shape=[
            jax.ShapeDtypeStruct((N, B), f32),
            jax.ShapeDtypeStruct((N, B), f32),
            jax.ShapeDtypeStruct((N, B), f32),
            jax.ShapeDtypeStruct((N, B), f32),
            jax.ShapeDtypeStruct((N, B), jnp.int32),
            jax.ShapeDtypeStruct((N, B), f32),
        ],
    )(s0, e0, c0, s1, e1, c1, xT)

    node_logits = jnp.stack([l0, l1], axis=-1)
    probs = jnp.stack([p0, p1], axis=-1)
    return node_logits, probs, preds, ent


# speedup vs baseline: 1.4637x; 1.0089x over previous
import jax
import jax.numpy as jnp
from jax.experimental import pallas as pl


def _tile_kernel(s0_ref, e0_ref, c0_ref, s1_ref, e1_ref, c1_ref, xT_ref,
                 l0_ref, l1_ref, p0_ref, p1_ref, pred_ref, ent_ref):
    bN = s0_ref.shape[0]
    C = xT_ref.shape[0]
    cls = jax.lax.broadcasted_iota(jnp.int32, (bN, C), 1)
    m0 = ((cls >= s0_ref[...]) & (cls < e0_ref[...])).astype(jnp.bfloat16)
    m1 = ((cls >= s1_ref[...]) & (cls < e1_ref[...])).astype(jnp.bfloat16)
    x = xT_ref[...]
    xhi = x.astype(jnp.bfloat16)
    xlo = (x - xhi.astype(jnp.float32)).astype(jnp.bfloat16)
    f32 = jnp.float32
    l0 = (jnp.dot(m0, xhi, preferred_element_type=f32)
          + jnp.dot(m0, xlo, preferred_element_type=f32)) / c0_ref[...]
    l1 = (jnp.dot(m1, xhi, preferred_element_type=f32)
          + jnp.dot(m1, xlo, preferred_element_type=f32)) / c1_ref[...]
    d = l1 - l0
    p0 = jax.nn.sigmoid(-d)
    p1 = jax.nn.sigmoid(d)
    l0_ref[...] = l0
    l1_ref[...] = l1
    p0_ref[...] = p0
    p1_ref[...] = p1
    pred_ref[...] = (d > 0).astype(jnp.int32)
    ent_ref[...] = -(p0 * jnp.log(p0) + p1 * jnp.log(p1))


def kernel(outputs, gather_idx, segment_ids, counts):
    B, C = outputs.shape
    S = counts.shape[0]
    N = S // 2
    del segment_ids

    cnt_i = counts.astype(jnp.int32)
    offsets = jnp.concatenate(
        [jnp.zeros((1,), jnp.int32), jnp.cumsum(cnt_i)[:-1]])
    starts = gather_idx[offsets]
    ends = starts + cnt_i

    s0 = starts[0::2][:, None]
    e0 = ends[0::2][:, None]
    s1 = starts[1::2][:, None]
    e1 = ends[1::2][:, None]
    c0 = counts[0::2][:, None]
    c1 = counts[1::2][:, None]

    xT = outputs.T

    bN = 512
    bB = 1024
    grid = (B // bB, pl.cdiv(N, bN))

    seg_spec = pl.BlockSpec((bN, 1), lambda j, i: (i, 0))
    out_spec = pl.BlockSpec((bN, bB), lambda j, i: (i, j))
    f32 = jnp.float32
    l0, l1, p0, p1, preds, ent = pl.pallas_call(
        _tile_kernel,
        grid=grid,
        in_specs=[seg_spec, seg_spec, seg_spec, seg_spec, seg_spec, seg_spec,
                  pl.BlockSpec((C, bB), lambda j, i: (0, j))],
        out_specs=[out_spec] * 6,
        out_shape=[
            jax.ShapeDtypeStruct((N, B), f32),
            jax.ShapeDtypeStruct((N, B), f32),
            jax.ShapeDtypeStruct((N, B), f32),
            jax.ShapeDtypeStruct((N, B), f32),
            jax.ShapeDtypeStruct((N, B), jnp.int32),
            jax.ShapeDtypeStruct((N, B), f32),
        ],
    )(s0, e0, c0, s1, e1, c1, xT)

    node_logits = jnp.stack([l0, l1], axis=-1)
    probs = jnp.stack([p0, p1], axis=-1)
    return node_logits, probs, preds, ent


# speedup vs baseline: 1.4918x; 1.0192x over previous
import jax
import jax.numpy as jnp
from jax.experimental import pallas as pl


def _tile_kernel(s0_ref, e0_ref, c0_ref, s1_ref, e1_ref, c1_ref, xT_ref,
                 l0_ref, l1_ref, p0_ref, pred_ref, ent_ref):
    bN = s0_ref.shape[0]
    C = xT_ref.shape[0]
    cls = jax.lax.broadcasted_iota(jnp.int32, (bN, C), 1)
    m0 = ((cls >= s0_ref[...]) & (cls < e0_ref[...])).astype(jnp.bfloat16)
    m1 = ((cls >= s1_ref[...]) & (cls < e1_ref[...])).astype(jnp.bfloat16)
    x = xT_ref[...]
    xhi = x.astype(jnp.bfloat16)
    xlo = (x - xhi.astype(jnp.float32)).astype(jnp.bfloat16)
    f32 = jnp.float32
    l0 = (jnp.dot(m0, xhi, preferred_element_type=f32)
          + jnp.dot(m0, xlo, preferred_element_type=f32)) / c0_ref[...]
    l1 = (jnp.dot(m1, xhi, preferred_element_type=f32)
          + jnp.dot(m1, xlo, preferred_element_type=f32)) / c1_ref[...]
    d = l1 - l0
    p0 = jax.nn.sigmoid(-d)
    p1 = jax.nn.sigmoid(d)
    l0_ref[...] = l0
    l1_ref[...] = l1
    p0_ref[...] = p0
    pred_ref[...] = (d > 0).astype(jnp.int32)
    ent_ref[...] = -(p0 * jnp.log(p0) + p1 * jnp.log(p1))


def kernel(outputs, gather_idx, segment_ids, counts):
    B, C = outputs.shape
    S = counts.shape[0]
    N = S // 2
    del segment_ids

    cnt_i = counts.astype(jnp.int32)
    offsets = jnp.concatenate(
        [jnp.zeros((1,), jnp.int32), jnp.cumsum(cnt_i)[:-1]])
    starts = gather_idx[offsets]
    ends = starts + cnt_i

    s0 = starts[0::2][:, None]
    e0 = ends[0::2][:, None]
    s1 = starts[1::2][:, None]
    e1 = ends[1::2][:, None]
    c0 = counts[0::2][:, None]
    c1 = counts[1::2][:, None]

    xT = outputs.T

    bN = 512
    bB = 1024
    grid = (B // bB, pl.cdiv(N, bN))

    seg_spec = pl.BlockSpec((bN, 1), lambda j, i: (i, 0))
    out_spec = pl.BlockSpec((bN, bB), lambda j, i: (i, j))
    f32 = jnp.float32
    l0, l1, p0, preds, ent = pl.pallas_call(
        _tile_kernel,
        grid=grid,
        in_specs=[seg_spec, seg_spec, seg_spec, seg_spec, seg_spec, seg_spec,
                  pl.BlockSpec((C, bB), lambda j, i: (0, j))],
        out_specs=[out_spec] * 5,
        out_shape=[
            jax.ShapeDtypeStruct((N, B), f32),
            jax.ShapeDtypeStruct((N, B), f32),
            jax.ShapeDtypeStruct((N, B), f32),
            jax.ShapeDtypeStruct((N, B), jnp.int32),
            jax.ShapeDtypeStruct((N, B), f32),
        ],
    )(s0, e0, c0, s1, e1, c1, xT)

    node_logits = jnp.stack([l0, l1], axis=-1)
    probs = jnp.stack([p0, 1.0 - p0], axis=-1)
    return node_logits, probs, preds, ent
